# Initial kernel scaffold; baseline (speedup 1.0000x reference)
import jax
import jax.numpy as jnp
from jax.experimental import pallas as pl


def kernel(inputs, roi_label, ious, lut, cq):
    raise NotImplementedError("write your pallas kernel here")


# speedup vs baseline: 5.5642x; 5.5642x over previous
import functools

import jax
import jax.numpy as jnp
from jax import lax
from jax.experimental import pallas as pl
from jax.experimental.pallas import tpu as pltpu
from jax.experimental.pallas import tpu_sc as plsc

N_ROWS = 1024
N_FEAT = 128
N_PIDS = 100000
N_CQ = 5000
SCALE = 30.0
IGNORE = 5554

TILE = 1000
NLUT = N_PIDS // TILE
NCQ = N_CQ // TILE


def _sumexp_body(x_ref, lut_ref, cq_ref, s_ref):
    i = pl.program_id(0)

    @pl.when(i == 0)
    def _init():
        s_ref[...] = jnp.zeros_like(s_ref)

    def acc(w):
        logits = lax.dot_general(
            x_ref[...], w.astype(jnp.bfloat16),
            (((1,), (1,)), ((), ())),
            preferred_element_type=jnp.float32,
        )
        s_ref[...] += jnp.sum(jnp.exp(logits), axis=1, keepdims=True)

    @pl.when(i < NLUT)
    def _lut():
        acc(lut_ref[...])

    @pl.when(i >= NLUT)
    def _cq():
        acc(cq_ref[...])


def _sumexp(x30_bf16, lut, cq):
    return pl.pallas_call(
        _sumexp_body,
        grid=(NLUT + NCQ,),
        in_specs=[
            pl.BlockSpec((N_ROWS, N_FEAT), lambda i: (0, 0)),
            pl.BlockSpec((TILE, N_FEAT), lambda i: (jnp.minimum(i, NLUT - 1), 0)),
            pl.BlockSpec((TILE, N_FEAT), lambda i: (jnp.maximum(i - NLUT, 0), 0)),
        ],
        out_specs=pl.BlockSpec((N_ROWS, 1), lambda i: (0, 0)),
        out_shape=jax.ShapeDtypeStruct((N_ROWS, 1), jnp.float32),
    )(x30_bf16, lut, cq)


_NW = 32
_BPW = N_ROWS // _NW


@functools.lru_cache(maxsize=1)
def _make_sc_gather():
    mesh = plsc.VectorSubcoreMesh(core_axis_name="c", subcore_axis_name="s")

    @functools.partial(
        pl.kernel,
        mesh=mesh,
        out_type=jax.ShapeDtypeStruct((N_ROWS, N_FEAT), jnp.float32),
        scratch_types=[
            pltpu.VMEM((_BPW,), jnp.int32),
            pltpu.VMEM((_BPW, N_FEAT), jnp.float32),
            pltpu.SemaphoreType.DMA,
        ],
    )
    def gather_k(table_hbm, idx_hbm, out_hbm, idx_v, rows_v, sem):
        wid = lax.axis_index("s") * 2 + lax.axis_index("c")
        base = wid * _BPW
        pltpu.sync_copy(idx_hbm.at[pl.ds(base, _BPW)], idx_v)
        pltpu.async_copy(table_hbm.at[idx_v], rows_v, sem).wait()
        pltpu.sync_copy(rows_v, out_hbm.at[pl.ds(base, _BPW)])

    return gather_k


def _combine_body(tgt_ref, x_ref, g_ref, s_ref, out_ref):
    label = tgt_ref[...] - 1
    keep = label >= 0
    valid = jnp.logical_and(keep, label != IGNORE)
    picked = SCALE * jnp.sum(x_ref[...] * g_ref[...], axis=1, keepdims=True)
    nll = jnp.log(s_ref[...]) - picked
    vm = valid.astype(jnp.float32)
    num = jnp.sum(nll * vm, keepdims=True)
    den = jnp.maximum(jnp.sum(vm, keepdims=True), 1.0)
    out_ref[...] = num / den


def _combine(tgt, x, g, s):
    return pl.pallas_call(
        _combine_body,
        out_shape=jax.ShapeDtypeStruct((1, 1), jnp.float32),
    )(tgt, x, g, s)


def kernel(inputs, roi_label, ious, lut, cq):
    tgt = roi_label.reshape(-1, 1).astype(jnp.int32)
    label = tgt[:, 0] - 1
    safe_label = jnp.where(label >= 0, label, 0).astype(jnp.int32)
    x30 = (inputs * SCALE).astype(jnp.bfloat16)

    g = _make_sc_gather()(lut, safe_label)
    s = _sumexp(x30, lut, cq)
    loss = _combine(tgt, inputs, g, s)
    return jnp.nan_to_num(loss.reshape(()))


# speedup vs baseline: 6.8591x; 1.2327x over previous
import functools

import jax
import jax.numpy as jnp
from jax import lax
from jax.experimental import pallas as pl
from jax.experimental.pallas import tpu as pltpu
from jax.experimental.pallas import tpu_sc as plsc

N_ROWS = 1024
N_FEAT = 128
N_PIDS = 100000
N_CQ = 5000
SCALE = 30.0
IGNORE = 5554

TILE_LUT = 2000
TILE_CQ = 1000


def _sumexp_body(x_ref, w_ref, s_ref):
    i = pl.program_id(0)

    @pl.when(i == 0)
    def _init():
        s_ref[...] = jnp.zeros_like(s_ref)

    t = lax.dot_general(
        x_ref[...], w_ref[...].astype(jnp.bfloat16),
        (((1,), (1,)), ((), ())),
        preferred_element_type=jnp.float32,
    )
    s_ref[...] += jnp.sum(jnp.exp2(t), axis=1, keepdims=True)


def _sumexp(xs_bf16, table, tile):
    n_rows = table.shape[0]
    return pl.pallas_call(
        _sumexp_body,
        grid=(n_rows // tile,),
        in_specs=[
            pl.BlockSpec((N_ROWS, N_FEAT), lambda i: (0, 0)),
            pl.BlockSpec((tile, N_FEAT), lambda i: (i, 0)),
        ],
        out_specs=pl.BlockSpec((N_ROWS, 1), lambda i: (0, 0)),
        out_shape=jax.ShapeDtypeStruct((N_ROWS, 1), jnp.float32),
    )(xs_bf16, table)


_NW = 32
_BPW = N_ROWS // _NW


@functools.lru_cache(maxsize=1)
def _make_sc_gather():
    mesh = plsc.VectorSubcoreMesh(core_axis_name="c", subcore_axis_name="s")

    @functools.partial(
        pl.kernel,
        mesh=mesh,
        out_type=jax.ShapeDtypeStruct((N_ROWS, N_FEAT), jnp.float32),
        scratch_types=[
            pltpu.VMEM((_BPW,), jnp.int32),
            pltpu.VMEM((_BPW, N_FEAT), jnp.float32),
            pltpu.SemaphoreType.DMA,
        ],
    )
    def gather_k(table_hbm, idx_hbm, out_hbm, idx_v, rows_v, sem):
        wid = lax.axis_index("s") * 2 + lax.axis_index("c")
        base = wid * _BPW
        pltpu.sync_copy(idx_hbm.at[pl.ds(base, _BPW)], idx_v)
        pltpu.async_copy(table_hbm.at[idx_v], rows_v, sem).wait()
        pltpu.sync_copy(rows_v, out_hbm.at[pl.ds(base, _BPW)])

    return gather_k


def _combine_body(tgt_ref, x_ref, g_ref, sa_ref, sb_ref, out_ref):
    label = tgt_ref[...] - 1
    keep = label >= 0
    valid = jnp.logical_and(keep, label != IGNORE)
    picked = SCALE * jnp.sum(x_ref[...] * g_ref[...], axis=1, keepdims=True)
    nll = jnp.log(sa_ref[...] + sb_ref[...]) - picked
    vm = valid.astype(jnp.float32)
    num = jnp.sum(nll * vm, keepdims=True)
    den = jnp.maximum(jnp.sum(vm, keepdims=True), 1.0)
    out_ref[...] = num / den


def _combine(tgt, x, g, sa, sb):
    return pl.pallas_call(
        _combine_body,
        out_shape=jax.ShapeDtypeStruct((1, 1), jnp.float32),
    )(tgt, x, g, sa, sb)


def kernel(inputs, roi_label, ious, lut, cq):
    tgt = roi_label.reshape(-1, 1).astype(jnp.int32)
    label = tgt[:, 0] - 1
    safe_label = jnp.where(label >= 0, label, 0).astype(jnp.int32)
    xs = (inputs * (SCALE * 1.4426950408889634)).astype(jnp.bfloat16)

    g = _make_sc_gather()(lut, safe_label)
    s_lut = _sumexp(xs, lut, TILE_LUT)
    s_cq = _sumexp(xs, cq, TILE_CQ)
    loss = _combine(tgt, inputs, g, s_lut, s_cq)
    return jnp.nan_to_num(loss.reshape(()))


# speedup vs baseline: 7.9086x; 1.1530x over previous
import functools

import jax
import jax.numpy as jnp
from jax import lax
from jax.experimental import pallas as pl
from jax.experimental.pallas import tpu as pltpu
from jax.experimental.pallas import tpu_sc as plsc

N_ROWS = 1024
N_FEAT = 128
N_PIDS = 100000
N_CQ = 5000
SCALE = 30.0
IGNORE = 5554
LOG2E = 1.4426950408889634

TILE_LUT = 4000


def _sumexp_body(x_ref, w_ref, s_ref):
    i = pl.program_id(0)

    @pl.when(i == 0)
    def _init():
        s_ref[...] = jnp.zeros_like(s_ref)

    xb = (x_ref[...] * (SCALE * LOG2E)).astype(jnp.bfloat16)
    t = lax.dot_general(
        xb, w_ref[...].astype(jnp.bfloat16),
        (((1,), (1,)), ((), ())),
        preferred_element_type=jnp.float32,
    )
    s_ref[...] += jnp.sum(jnp.exp2(t), axis=1, keepdims=True)


def _sumexp_lut(x, lut):
    return pl.pallas_call(
        _sumexp_body,
        grid=(N_PIDS // TILE_LUT,),
        in_specs=[
            pl.BlockSpec((N_ROWS, N_FEAT), lambda i: (0, 0)),
            pl.BlockSpec((TILE_LUT, N_FEAT), lambda i: (i, 0)),
        ],
        out_specs=pl.BlockSpec((N_ROWS, 1), lambda i: (0, 0)),
        out_shape=jax.ShapeDtypeStruct((N_ROWS, 1), jnp.float32),
    )(x, lut)


_NW = 32
_BPW = N_ROWS // _NW


@functools.lru_cache(maxsize=1)
def _make_sc_gather():
    mesh = plsc.VectorSubcoreMesh(core_axis_name="c", subcore_axis_name="s")

    @functools.partial(
        pl.kernel,
        mesh=mesh,
        out_type=jax.ShapeDtypeStruct((N_ROWS, N_FEAT), jnp.float32),
        scratch_types=[
            pltpu.VMEM((_BPW,), jnp.int32),
            pltpu.VMEM((_BPW, N_FEAT), jnp.float32),
            pltpu.SemaphoreType.DMA,
        ],
    )
    def gather_k(table_hbm, idx_hbm, out_hbm, idx_v, rows_v, sem):
        wid = lax.axis_index("s") * 2 + lax.axis_index("c")
        base = wid * _BPW
        pltpu.sync_copy(idx_hbm.at[pl.ds(base, _BPW)], idx_v)
        pltpu.async_copy(table_hbm.at[idx_v], rows_v, sem).wait()
        pltpu.sync_copy(rows_v, out_hbm.at[pl.ds(base, _BPW)])

    return gather_k


def _combine_body(tgt_ref, x_ref, g_ref, cq_ref, sa_ref, out_ref):
    x = x_ref[...]
    xb = (x * (SCALE * LOG2E)).astype(jnp.bfloat16)
    t = lax.dot_general(
        xb, cq_ref[...].astype(jnp.bfloat16),
        (((1,), (1,)), ((), ())),
        preferred_element_type=jnp.float32,
    )
    s = sa_ref[...] + jnp.sum(jnp.exp2(t), axis=1, keepdims=True)

    label = tgt_ref[...] - 1
    keep = label >= 0
    valid = jnp.logical_and(keep, label != IGNORE)
    picked = SCALE * jnp.sum(x * g_ref[...], axis=1, keepdims=True)
    nll = jnp.log(s) - picked
    vm = valid.astype(jnp.float32)
    num = jnp.sum(nll * vm, keepdims=True)
    den = jnp.maximum(jnp.sum(vm, keepdims=True), 1.0)
    out_ref[...] = num / den


def _combine(tgt, x, g, cq, sa):
    return pl.pallas_call(
        _combine_body,
        out_shape=jax.ShapeDtypeStruct((1, 1), jnp.float32),
    )(tgt, x, g, cq, sa)


def kernel(inputs, roi_label, ious, lut, cq):
    tgt = roi_label.reshape(-1, 1).astype(jnp.int32)
    label = tgt[:, 0] - 1
    safe_label = jnp.where(label >= 0, label, 0).astype(jnp.int32)

    g = _make_sc_gather()(lut, safe_label)
    s_lut = _sumexp_lut(inputs, lut)
    loss = _combine(tgt, inputs, g, cq, s_lut)
    return jnp.nan_to_num(loss.reshape(()))
